# baseline (speedup 1.0000x reference)
import jax
import jax.numpy as jnp
from jax import lax
from jax.experimental import pallas as pl
from jax.experimental.pallas import tpu as pltpu
from jax.experimental.pallas import tpu_sc as plsc

MAX_NUM_PAGES = 2048
PAGE_SIZE = 16
N_HEADS = 8
HEAD_DIM = 128
T = 128
NROWS = MAX_NUM_PAGES * 2 * PAGE_SIZE
L = 16
NVREG = T // L
NWORKERS = 32
STRIPE = NROWS // NWORKERS
CH = 32
NCH = STRIPE // CH


def _sc_body(k_hbm, v_hbm, a_lo_hbm, a_hi_hbm, p_lo_hbm, p_hi_hbm,
             ll_hbm, pidx_hbm, cache_hbm, out_hbm,
             a_lo_v, a_hi_v, p_lo_v, p_hi_v, ll_v, pidx_v, buf,
             in_sem0, in_sem1, out_sem0, out_sem1, tok_sem):
  wid = lax.axis_index("s") * 2 + lax.axis_index("c")
  base_r = wid * STRIPE
  in_sems = (in_sem0, in_sem1)
  out_sems = (out_sem0, out_sem1)

  def start_in(c):
    return pltpu.async_copy(
        cache_hbm.at[pl.ds(base_r + c * CH, CH)], buf.at[c % 2],
        in_sems[c % 2])

  def start_out(c):
    return pltpu.async_copy(
        buf.at[c % 2], out_hbm.at[pl.ds(base_r + c * CH, CH)],
        out_sems[c % 2])

  in_d = [None] * NCH
  out_d = [None] * NCH
  in_d[0] = start_in(0)

  pltpu.sync_copy(a_lo_hbm, a_lo_v)
  pltpu.sync_copy(a_hi_hbm, a_hi_v)
  pltpu.sync_copy(p_lo_hbm, p_lo_v)
  pltpu.sync_copy(p_hi_hbm, p_hi_v)
  pltpu.sync_copy(ll_hbm, ll_v)
  pltpu.sync_copy(pidx_hbm, pidx_v)

  dest_regs = []
  for g in range(NVREG):
    t = lax.iota(jnp.int32, L) + (g * L)
    a_lo = a_lo_v[pl.ds(g * L, L)]
    a_hi = a_hi_v[pl.ds(g * L, L)]
    p_lo = p_lo_v[pl.ds(g * L, L)]
    p_hi = p_hi_v[pl.ds(g * L, L)]
    ll = ll_v[pl.ds(g * L, L)]
    j = t - a_lo
    append_len = a_hi - a_lo
    n_pages = p_hi - p_lo
    kv_len = (n_pages - 1) * PAGE_SIZE + ll
    pos = kv_len - append_len + j
    page_local = lax.shift_right_arithmetic(pos, 4)
    slot = lax.bitwise_and(pos, PAGE_SIZE - 1)
    page = plsc.load_gather(pidx_v, [p_lo + page_local])
    dest_regs.append(page * (2 * PAGE_SIZE) + slot)

  for c in range(NCH):
    if c >= 1:
      out_d[c - 1].wait()
    if c + 1 < NCH:
      in_d[c + 1] = start_in(c + 1)
    in_d[c].wait()
    out_d[c] = start_out(c)
  out_d[NCH - 1].wait()

  for u in range(T):
    d = dest_regs[u // L][u % L]

    @pl.when(jnp.logical_and(d >= base_r, d < base_r + STRIPE))
    def _write(u=u, d=d):
      pltpu.async_copy(k_hbm.at[u], out_hbm.at[d], tok_sem).wait()
      pltpu.async_copy(v_hbm.at[u], out_hbm.at[d + PAGE_SIZE], tok_sem).wait()


_sc_append = pl.kernel(
    _sc_body,
    out_type=jax.ShapeDtypeStruct((NROWS, N_HEADS, HEAD_DIM), jnp.float32),
    mesh=plsc.VectorSubcoreMesh(core_axis_name="c", subcore_axis_name="s"),
    compiler_params=pltpu.CompilerParams(needs_layout_passes=False),
    scratch_types=[
        pltpu.VMEM((T,), jnp.int32),
        pltpu.VMEM((T,), jnp.int32),
        pltpu.VMEM((T,), jnp.int32),
        pltpu.VMEM((T,), jnp.int32),
        pltpu.VMEM((T,), jnp.int32),
        pltpu.VMEM((T,), jnp.int32),
        pltpu.VMEM((2, CH, N_HEADS, HEAD_DIM), jnp.float32),
        pltpu.SemaphoreType.DMA,
        pltpu.SemaphoreType.DMA,
        pltpu.SemaphoreType.DMA,
        pltpu.SemaphoreType.DMA,
        pltpu.SemaphoreType.DMA,
    ],
    name="paged_kv_append",
)


def kernel(k, v, kv_append_indptr, kv_cache, kv_page_indices, kv_page_indptr,
           kv_page_lastlen):
  a_lo = kv_append_indptr[:T]
  a_hi = kv_append_indptr[1:T + 1]
  p_lo = kv_page_indptr[:T]
  p_hi = kv_page_indptr[1:T + 1]
  out = _sc_append(k, v, a_lo, a_hi, p_lo, p_hi, kv_page_lastlen,
                   kv_page_indices, kv_cache.reshape(NROWS, N_HEADS, HEAD_DIM))
  return out.reshape(kv_cache.shape)


# speedup vs baseline: 1.0554x; 1.0554x over previous
import jax
import jax.numpy as jnp
from jax import lax
from jax.experimental import pallas as pl
from jax.experimental.pallas import tpu as pltpu
from jax.experimental.pallas import tpu_sc as plsc

MAX_NUM_PAGES = 2048
PAGE_SIZE = 16
N_HEADS = 8
HEAD_DIM = 128
T = 128
NROWS = MAX_NUM_PAGES * 2 * PAGE_SIZE
L = 16
NVREG = T // L
NWORKERS = 32
STRIPE = NROWS // NWORKERS
CH = 32
NCH = STRIPE // CH


def _sc_body(k_hbm, v_hbm, a_lo_hbm, a_hi_hbm, p_lo_hbm, p_hi_hbm,
             ll_hbm, pidx_hbm, cache_hbm, out_hbm,
             a_lo_v, a_hi_v, p_lo_v, p_hi_v, ll_v, pidx_v, buf,
             in_sem0, in_sem1, out_sem0, out_sem1, tok_sem):
  sid = lax.axis_index("s")
  wid = sid * 2 + lax.axis_index("c")
  base_r = wid * STRIPE
  in_sems = (in_sem0, in_sem1)
  out_sems = (out_sem0, out_sem1)

  def start_in(c):
    return pltpu.async_copy(
        cache_hbm.at[pl.ds(base_r + c * CH, CH)], buf.at[sid, c % 2],
        in_sems[c % 2])

  def start_out(c):
    return pltpu.async_copy(
        buf.at[sid, c % 2], out_hbm.at[pl.ds(base_r + c * CH, CH)],
        out_sems[c % 2])

  in_d = [None] * NCH
  out_d = [None] * NCH
  in_d[0] = start_in(0)

  pltpu.sync_copy(a_lo_hbm, a_lo_v)
  pltpu.sync_copy(a_hi_hbm, a_hi_v)
  pltpu.sync_copy(p_lo_hbm, p_lo_v)
  pltpu.sync_copy(p_hi_hbm, p_hi_v)
  pltpu.sync_copy(ll_hbm, ll_v)
  pltpu.sync_copy(pidx_hbm, pidx_v)

  dest_regs = []
  for g in range(NVREG):
    t = lax.iota(jnp.int32, L) + (g * L)
    a_lo = a_lo_v[pl.ds(g * L, L)]
    a_hi = a_hi_v[pl.ds(g * L, L)]
    p_lo = p_lo_v[pl.ds(g * L, L)]
    p_hi = p_hi_v[pl.ds(g * L, L)]
    ll = ll_v[pl.ds(g * L, L)]
    j = t - a_lo
    append_len = a_hi - a_lo
    n_pages = p_hi - p_lo
    kv_len = (n_pages - 1) * PAGE_SIZE + ll
    pos = kv_len - append_len + j
    page_local = lax.shift_right_arithmetic(pos, 4)
    slot = lax.bitwise_and(pos, PAGE_SIZE - 1)
    page = plsc.load_gather(pidx_v, [p_lo + page_local])
    dest_regs.append(page * (2 * PAGE_SIZE) + slot)

  for c in range(NCH):
    if c >= 1:
      out_d[c - 1].wait()
    if c + 1 < NCH:
      in_d[c + 1] = start_in(c + 1)
    in_d[c].wait()
    out_d[c] = start_out(c)
  out_d[NCH - 1].wait()

  for u in range(T):
    d = dest_regs[u // L][u % L]

    @pl.when(jnp.logical_and(d >= base_r, d < base_r + STRIPE))
    def _write(u=u, d=d):
      pltpu.async_copy(k_hbm.at[u], out_hbm.at[d], tok_sem).wait()
      pltpu.async_copy(v_hbm.at[u], out_hbm.at[d + PAGE_SIZE], tok_sem).wait()


_sc_append = pl.kernel(
    _sc_body,
    out_type=jax.ShapeDtypeStruct((NROWS, N_HEADS, HEAD_DIM), jnp.float32),
    mesh=plsc.VectorSubcoreMesh(core_axis_name="c", subcore_axis_name="s"),
    compiler_params=pltpu.CompilerParams(needs_layout_passes=False),
    scratch_types=[
        pltpu.VMEM((T,), jnp.int32),
        pltpu.VMEM((T,), jnp.int32),
        pltpu.VMEM((T,), jnp.int32),
        pltpu.VMEM((T,), jnp.int32),
        pltpu.VMEM((T,), jnp.int32),
        pltpu.VMEM((T,), jnp.int32),
        pltpu.VMEM_SHARED((16, 2, CH, N_HEADS, HEAD_DIM),
                          jnp.float32),
        pltpu.SemaphoreType.DMA,
        pltpu.SemaphoreType.DMA,
        pltpu.SemaphoreType.DMA,
        pltpu.SemaphoreType.DMA,
        pltpu.SemaphoreType.DMA,
    ],
    name="paged_kv_append",
)


def kernel(k, v, kv_append_indptr, kv_cache, kv_page_indices, kv_page_indptr,
           kv_page_lastlen):
  a_lo = kv_append_indptr[:T]
  a_hi = kv_append_indptr[1:T + 1]
  p_lo = kv_page_indptr[:T]
  p_hi = kv_page_indptr[1:T + 1]
  out = _sc_append(k, v, a_lo, a_hi, p_lo, p_hi, kv_page_lastlen,
                   kv_page_indices, kv_cache.reshape(NROWS, N_HEADS, HEAD_DIM))
  return out.reshape(kv_cache.shape)


# speedup vs baseline: 1.2992x; 1.2311x over previous
import jax
import jax.numpy as jnp
from jax import lax
from jax.experimental import pallas as pl
from jax.experimental.pallas import tpu as pltpu
from jax.experimental.pallas import tpu_sc as plsc

MAX_NUM_PAGES = 2048
PAGE_SIZE = 16
N_HEADS = 8
HEAD_DIM = 128
T = 128
ROW = N_HEADS * HEAD_DIM
NROWS = MAX_NUM_PAGES * 2 * PAGE_SIZE
L = 16
NVREG = T // L


def _sc_body(k_hbm, v_hbm, a_lo_hbm, a_hi_hbm, p_lo_hbm, p_hi_hbm,
             ll_hbm, pidx_hbm, cache_hbm,
             a_lo_v, a_hi_v, p_lo_v, p_hi_v, ll_v, pidx_v,
             dest_v, src_idx_v, dst_idx_v, buf_v, sem):
  wid = lax.axis_index("s") * 2 + lax.axis_index("c")

  @pl.when(wid < 2 * NVREG)
  def _work():
    gsub = lax.rem(wid, NVREG)
    kv_sel = wid // NVREG

    pltpu.sync_copy(a_lo_hbm, a_lo_v)
    pltpu.sync_copy(a_hi_hbm, a_hi_v)
    pltpu.sync_copy(p_lo_hbm, p_lo_v)
    pltpu.sync_copy(p_hi_hbm, p_hi_v)
    pltpu.sync_copy(ll_hbm, ll_v)
    pltpu.sync_copy(pidx_hbm, pidx_v)

    for g in range(NVREG):
      t = lax.iota(jnp.int32, L) + (g * L)
      a_lo = a_lo_v[pl.ds(g * L, L)]
      a_hi = a_hi_v[pl.ds(g * L, L)]
      p_lo = p_lo_v[pl.ds(g * L, L)]
      p_hi = p_hi_v[pl.ds(g * L, L)]
      ll = ll_v[pl.ds(g * L, L)]
      j = t - a_lo
      append_len = a_hi - a_lo
      n_pages = p_hi - p_lo
      kv_len = (n_pages - 1) * PAGE_SIZE + ll
      pos = kv_len - append_len + j
      page_local = lax.shift_right_arithmetic(pos, 4)
      slot = lax.bitwise_and(pos, PAGE_SIZE - 1)
      page = plsc.load_gather(pidx_v, [p_lo + page_local])
      dest_v[pl.ds(g * L, L)] = page * (2 * PAGE_SIZE) + slot

    base = lax.mul(gsub, L)
    my_t = lax.iota(jnp.int32, L) + base
    my_dest = dest_v[pl.ds(base, L)]

    w = my_t
    for u in range(T):
      bc = plsc.load_gather(dest_v, [jnp.full((L,), u, jnp.int32)])
      w = jnp.where(my_dest == bc, jnp.int32(u), w)

    src_idx_v[...] = w
    dst_idx_v[...] = my_dest + kv_sel * PAGE_SIZE

    @pl.when(kv_sel == 0)
    def _gather_k():
      pltpu.async_copy(k_hbm.at[src_idx_v], buf_v, sem).wait()

    @pl.when(kv_sel == 1)
    def _gather_v():
      pltpu.async_copy(v_hbm.at[src_idx_v], buf_v, sem).wait()

    pltpu.async_copy(buf_v, cache_hbm.at[dst_idx_v], sem).wait()


_sc_scatter = pl.kernel(
    _sc_body,
    out_type=(),
    mesh=plsc.VectorSubcoreMesh(core_axis_name="c", subcore_axis_name="s"),
    compiler_params=pltpu.CompilerParams(needs_layout_passes=False),
    scratch_types=[
        pltpu.VMEM((T,), jnp.int32),
        pltpu.VMEM((T,), jnp.int32),
        pltpu.VMEM((T,), jnp.int32),
        pltpu.VMEM((T,), jnp.int32),
        pltpu.VMEM((T,), jnp.int32),
        pltpu.VMEM((T,), jnp.int32),
        pltpu.VMEM((T,), jnp.int32),
        pltpu.VMEM((L,), jnp.int32),
        pltpu.VMEM((L,), jnp.int32),
        pltpu.VMEM((L, N_HEADS, HEAD_DIM), jnp.float32),
        pltpu.SemaphoreType.DMA,
    ],
    name="paged_kv_append_scatter",
)


def kernel(k, v, kv_append_indptr, kv_cache, kv_page_indices, kv_page_indptr,
           kv_page_lastlen):
  a_lo = kv_append_indptr[:T]
  a_hi = kv_append_indptr[1:T + 1]
  p_lo = kv_page_indptr[:T]
  p_hi = kv_page_indptr[1:T + 1]
  cache_ref = jax.new_ref(kv_cache.reshape(NROWS, N_HEADS, HEAD_DIM))
  _sc_scatter(k, v, a_lo, a_hi, p_lo, p_hi, kv_page_lastlen,
              kv_page_indices, cache_ref)
  return jax.ref.freeze(cache_ref).reshape(kv_cache.shape)


# speedup vs baseline: 1.3159x; 1.0129x over previous
import jax
import jax.numpy as jnp
from jax import lax
from jax.experimental import pallas as pl
from jax.experimental.pallas import tpu as pltpu
from jax.experimental.pallas import tpu_sc as plsc

MAX_NUM_PAGES = 2048
PAGE_SIZE = 16
N_HEADS = 8
HEAD_DIM = 128
T = 128
ROW = N_HEADS * HEAD_DIM
NROWS = MAX_NUM_PAGES * 2 * PAGE_SIZE
L = 16
NVREG = T // L


def _sc_body(k_hbm, v_hbm, a_lo_hbm, a_hi_hbm, p_lo_hbm, p_hi_hbm,
             ll_hbm, pidx_hbm, cache_hbm,
             a_lo_v, a_hi_v, p_lo_v, p_hi_v, ll_v, pidx_v,
             dest_v, src_idx_v, dst_idx_v, buf_v, sem):
  wid = lax.axis_index("s") * 2 + lax.axis_index("c")

  @pl.when(wid < 2 * NVREG)
  def _work():
    gsub = lax.rem(wid, NVREG)
    kv_sel = wid // NVREG

    stages = [pltpu.async_copy(s, d, sem) for s, d in (
        (a_lo_hbm, a_lo_v), (a_hi_hbm, a_hi_v), (p_lo_hbm, p_lo_v),
        (p_hi_hbm, p_hi_v), (ll_hbm, ll_v), (pidx_hbm, pidx_v))]
    for s in stages:
      s.wait()

    dest_regs = []
    for g in range(NVREG):
      t = lax.iota(jnp.int32, L) + (g * L)
      a_lo = a_lo_v[pl.ds(g * L, L)]
      a_hi = a_hi_v[pl.ds(g * L, L)]
      p_lo = p_lo_v[pl.ds(g * L, L)]
      p_hi = p_hi_v[pl.ds(g * L, L)]
      ll = ll_v[pl.ds(g * L, L)]
      j = t - a_lo
      append_len = a_hi - a_lo
      n_pages = p_hi - p_lo
      kv_len = (n_pages - 1) * PAGE_SIZE + ll
      pos = kv_len - append_len + j
      page_local = lax.shift_right_arithmetic(pos, 4)
      slot = lax.bitwise_and(pos, PAGE_SIZE - 1)
      page = plsc.load_gather(pidx_v, [p_lo + page_local])
      dest = page * (2 * PAGE_SIZE) + slot
      dest_regs.append(dest)
      dest_v[pl.ds(g * L, L)] = dest

    base = lax.mul(gsub, L)
    my_t = lax.iota(jnp.int32, L) + base
    my_dest = dest_v[pl.ds(base, L)]

    w = my_t
    for u in range(T):
      bc = jnp.full((L,), dest_regs[u // L][u % L], jnp.int32)
      w = jnp.where(my_dest == bc, jnp.int32(u), w)

    src_idx_v[...] = w
    dst_idx_v[...] = my_dest + kv_sel * PAGE_SIZE

    @pl.when(kv_sel == 0)
    def _gather_k():
      pltpu.async_copy(k_hbm.at[src_idx_v], buf_v, sem).wait()

    @pl.when(kv_sel == 1)
    def _gather_v():
      pltpu.async_copy(v_hbm.at[src_idx_v], buf_v, sem).wait()

    pltpu.async_copy(buf_v, cache_hbm.at[dst_idx_v], sem).wait()


_sc_scatter = pl.kernel(
    _sc_body,
    out_type=(),
    mesh=plsc.VectorSubcoreMesh(core_axis_name="c", subcore_axis_name="s"),
    compiler_params=pltpu.CompilerParams(needs_layout_passes=False),
    scratch_types=[
        pltpu.VMEM((T,), jnp.int32),
        pltpu.VMEM((T,), jnp.int32),
        pltpu.VMEM((T,), jnp.int32),
        pltpu.VMEM((T,), jnp.int32),
        pltpu.VMEM((T,), jnp.int32),
        pltpu.VMEM((T,), jnp.int32),
        pltpu.VMEM((T,), jnp.int32),
        pltpu.VMEM((L,), jnp.int32),
        pltpu.VMEM((L,), jnp.int32),
        pltpu.VMEM((L, N_HEADS, HEAD_DIM), jnp.float32),
        pltpu.SemaphoreType.DMA,
    ],
    name="paged_kv_append_scatter",
)


def kernel(k, v, kv_append_indptr, kv_cache, kv_page_indices, kv_page_indptr,
           kv_page_lastlen):
  a_lo = kv_append_indptr[:T]
  a_hi = kv_append_indptr[1:T + 1]
  p_lo = kv_page_indptr[:T]
  p_hi = kv_page_indptr[1:T + 1]
  cache_ref = jax.new_ref(kv_cache.reshape(NROWS, N_HEADS, HEAD_DIM))
  _sc_scatter(k, v, a_lo, a_hi, p_lo, p_hi, kv_page_lastlen,
              kv_page_indices, cache_ref)
  return jax.ref.freeze(cache_ref).reshape(kv_cache.shape)
